# baseline (speedup 1.0000x reference)
import functools

import jax
import jax.numpy as jnp
from jax import lax
from jax.experimental import pallas as pl
from jax.experimental.pallas import tpu as pltpu
from jax.experimental.pallas import tpu_sc as plsc

_B, _S, _H = 1024, 200, 128
_N = _B * _S
_NC, _NS = 2, 16
_NW = _NC * _NS
_PER_W = _N // _NW
_CH = 64
_NCH = _PER_W // _CH
_NBUF = 8
_GDEPTH = 6

_mesh = plsc.VectorSubcoreMesh(core_axis_name="c", subcore_axis_name="s")


@functools.partial(
    pl.kernel,
    out_type=jax.ShapeDtypeStruct((_N, _H), jnp.float32),
    mesh=_mesh,
    scratch_types=[
        pltpu.VMEM((_NCH, _CH), jnp.int32),
        pltpu.VMEM((_NBUF, _CH, _H), jnp.float32),
        pltpu.SemaphoreType.DMA,
        pltpu.SemaphoreType.DMA,
    ],
)
def _embed_gather(idx_hbm, tbl_hbm, out_hbm, idx_v, rows_v, gsem, wsem):
    wid = lax.axis_index("s") * _NC + lax.axis_index("c")
    row0 = wid * _PER_W
    pltpu.sync_copy(idx_hbm.at[wid], idx_v)

    def fire(s, p):
        pltpu.async_copy(tbl_hbm.at[idx_v.at[s]], rows_v.at[p], gsem)

    def drain_gather(p):
        pltpu.make_async_copy(
            tbl_hbm.at[pl.ds(0, _CH)], rows_v.at[p], gsem
        ).wait()

    def start_wb(s, p):
        pltpu.async_copy(
            rows_v.at[p], out_hbm.at[pl.ds(row0 + s * _CH, _CH)], wsem
        )

    def drain_wb():
        pltpu.make_async_copy(
            rows_v.at[0], out_hbm.at[pl.ds(0, _CH)], wsem
        ).wait()

    def stage(s, p, wbwait, nfire):
        drain_gather(p)
        if wbwait:
            drain_wb()
        start_wb(s, p)
        if nfire:
            fire(s + _GDEPTH, (p + _GDEPTH) % _NBUF)

    for s in range(_GDEPTH):
        fire(s, s)
    stage(0, 0, wbwait=False, nfire=True)
    stage(1, 1, wbwait=False, nfire=True)

    @pl.loop(2, 90, step=_NBUF)
    def _steady(g):
        for i in range(_NBUF):
            stage(g + i, (2 + i) % _NBUF, wbwait=True, nfire=True)

    for s in range(90, 94):
        stage(s, s % _NBUF, wbwait=True, nfire=True)
    for s in range(94, 100):
        stage(s, s % _NBUF, wbwait=True, nfire=False)
    drain_wb()
    drain_wb()


def kernel(input_ids, token_table):
    idx = input_ids.reshape(_NW, _NCH, _CH).astype(jnp.int32)
    out = _embed_gather(idx, token_table)
    return out.reshape(_B, _S, _H)


# speedup vs baseline: 1.0114x; 1.0114x over previous
import functools

import jax
import jax.numpy as jnp
from jax import lax
from jax.experimental import pallas as pl
from jax.experimental.pallas import tpu as pltpu
from jax.experimental.pallas import tpu_sc as plsc

_B, _S, _H = 1024, 200, 128
_N = _B * _S
_NC, _NS = 2, 16
_NW = _NC * _NS
_PER_W = _N // _NW
_CH = 128
_NCH = _PER_W // _CH
_NBUF = 6
_GDEPTH = 4

_mesh = plsc.VectorSubcoreMesh(core_axis_name="c", subcore_axis_name="s")


@functools.partial(
    pl.kernel,
    out_type=jax.ShapeDtypeStruct((_N, _H), jnp.float32),
    mesh=_mesh,
    scratch_types=[
        pltpu.VMEM((_NCH, _CH), jnp.int32),
        pltpu.VMEM((_NBUF, _CH, _H), jnp.float32),
        pltpu.SemaphoreType.DMA,
        pltpu.SemaphoreType.DMA,
    ],
)
def _embed_gather(idx_hbm, tbl_hbm, out_hbm, idx_v, rows_v, gsem, wsem):
    wid = lax.axis_index("s") * _NC + lax.axis_index("c")
    row0 = wid * _PER_W
    pltpu.sync_copy(idx_hbm.at[wid], idx_v)

    def fire(s, p):
        pltpu.async_copy(tbl_hbm.at[idx_v.at[s]], rows_v.at[p], gsem)

    def drain_gather(p):
        pltpu.make_async_copy(
            tbl_hbm.at[pl.ds(0, _CH)], rows_v.at[p], gsem
        ).wait()

    def start_wb(s, p):
        pltpu.async_copy(
            rows_v.at[p], out_hbm.at[pl.ds(row0 + s * _CH, _CH)], wsem
        )

    def drain_wb():
        pltpu.make_async_copy(
            rows_v.at[0], out_hbm.at[pl.ds(0, _CH)], wsem
        ).wait()

    def stage(s, p, wbwait, nfire):
        drain_gather(p)
        if wbwait:
            drain_wb()
        start_wb(s, p)
        if nfire:
            fire(s + _GDEPTH, (p + _GDEPTH) % _NBUF)

    for s in range(_GDEPTH):
        fire(s, s)
    stage(0, 0, wbwait=False, nfire=True)
    stage(1, 1, wbwait=False, nfire=True)

    @pl.loop(2, 44, step=_NBUF)
    def _steady(g):
        for i in range(_NBUF):
            stage(g + i, (2 + i) % _NBUF, wbwait=True, nfire=True)

    for s in range(44, 46):
        stage(s, s % _NBUF, wbwait=True, nfire=True)
    for s in range(46, 50):
        stage(s, s % _NBUF, wbwait=True, nfire=False)
    drain_wb()
    drain_wb()


def kernel(input_ids, token_table):
    idx = input_ids.reshape(_NW, _NCH, _CH).astype(jnp.int32)
    out = _embed_gather(idx, token_table)
    return out.reshape(_B, _S, _H)
